# Initial kernel scaffold; baseline (speedup 1.0000x reference)
import functools

import jax
import jax.numpy as jnp
from jax import lax
from jax.experimental import pallas as pl
from jax.experimental.pallas import tpu as pltpu
from jax.experimental.pallas import tpu_sc as plsc

BATCH = 4096
HIST = 50
EMBED = 64
NC = 2
NS = 16
NW = NC * NS
B = BATCH * HIST
BPW = B // NW
CHUNK = 128
NCHUNK = BPW // CHUNK


def _emb_body(idx_hbm, table_hbm, out_hbm, idx_v, rows_v, sem):
    wid = lax.axis_index("s") * NC + lax.axis_index("c")
    base = wid * BPW
    pltpu.sync_copy(idx_hbm.at[wid], idx_v)

    def body(ch, _):
        pltpu.async_copy(table_hbm.at[idx_v.at[ch]], rows_v, sem).wait()
        pltpu.sync_copy(rows_v, out_hbm.at[pl.ds(base + ch * CHUNK, CHUNK)])
        return 0

    lax.fori_loop(0, NCHUNK, body, 0)


@jax.jit
def _emb(idx, table):
    mesh = plsc.VectorSubcoreMesh(core_axis_name="c", subcore_axis_name="s")
    f = functools.partial(
        pl.kernel,
        mesh=mesh,
        out_type=jax.ShapeDtypeStruct((B, EMBED), jnp.float32),
        scratch_types=[
            pltpu.VMEM((NCHUNK, CHUNK), jnp.int32),
            pltpu.VMEM((CHUNK, EMBED), jnp.float32),
            pltpu.SemaphoreType.DMA,
        ],
    )(_emb_body)
    return f(idx, table)


def kernel(indices, table):
    idx = indices.reshape(NW, NCHUNK, CHUNK)
    out = _emb(idx, table)
    return out.reshape(BATCH, HIST, EMBED)


# speedup vs baseline: 1.1040x; 1.1040x over previous
import functools

import jax
import jax.numpy as jnp
from jax import lax
from jax.experimental import pallas as pl
from jax.experimental.pallas import tpu as pltpu
from jax.experimental.pallas import tpu_sc as plsc

BATCH = 4096
HIST = 50
EMBED = 64
NC = 2
NS = 16
NW = NC * NS
B = BATCH * HIST
BPW = B // NW
CHUNK = 128
NCHUNK = BPW // CHUNK


def _emb_body(idx_hbm, table_hbm, out_hbm, idx_v, rows_v, sem):
    wid = lax.axis_index("s") * NC + lax.axis_index("c")
    base = wid * BPW
    pltpu.sync_copy(idx_hbm.at[wid], idx_v)

    def body(ch, _):
        pltpu.async_copy(table_hbm.at[idx_v.at[ch]], rows_v, sem).wait()
        pltpu.sync_copy(rows_v, out_hbm.at[pl.ds(base + ch * CHUNK, CHUNK)])
        return 0

    lax.fori_loop(0, NCHUNK, body, 0)


@jax.jit
def _emb(idx, table):
    mesh = plsc.VectorSubcoreMesh(core_axis_name="c", subcore_axis_name="s")
    f = functools.partial(
        pl.kernel,
        mesh=mesh,
        out_type=jax.ShapeDtypeStruct((B, EMBED), jnp.float32),
        scratch_types=[
            pltpu.VMEM((NCHUNK, CHUNK), jnp.int32),
            pltpu.VMEM((CHUNK, EMBED), jnp.float32),
            pltpu.SemaphoreType.DMA,
        ],
        compiler_params=pltpu.CompilerParams(use_tc_tiling_on_sc=False),
    )(_emb_body)
    return f(idx, table)


def kernel(indices, table):
    idx = indices.reshape(NW, NCHUNK, CHUNK)
    out = _emb(idx, table)
    return out.reshape(BATCH, HIST, EMBED)


# speedup vs baseline: 1.1550x; 1.0462x over previous
import functools

import jax
import jax.numpy as jnp
from jax import lax
from jax.experimental import pallas as pl
from jax.experimental.pallas import tpu as pltpu
from jax.experimental.pallas import tpu_sc as plsc

BATCH = 4096
HIST = 50
EMBED = 64
NC = 2
NS = 16
NW = NC * NS
B = BATCH * HIST
BPW = B // NW
CHUNK = 128
NCHUNK = BPW // CHUNK
K = 5
ROWS_R = K * CHUNK
ROUNDS = NCHUNK // K


def _emb_body(idx_hbm, table_hbm, out_hbm, idx_v, rows_v, sem_g, sem_s):
    wid = lax.axis_index("s") * NC + lax.axis_index("c")
    base = wid * BPW
    pltpu.sync_copy(idx_hbm.at[wid], idx_v)

    def fire_gathers(r, buf):
        for k in range(K):
            pltpu.async_copy(
                table_hbm.at[idx_v.at[r * K + k]],
                rows_v.at[buf, pl.ds(k * CHUNK, CHUNK)],
                sem_g.at[buf],
            )

    def drain_gathers(buf):
        for k in range(K):
            pltpu.make_async_copy(
                table_hbm.at[idx_v.at[0]],
                rows_v.at[buf, pl.ds(k * CHUNK, CHUNK)],
                sem_g.at[buf],
            ).wait()

    fire_gathers(0, 0)

    def round_step(r, buf):
        other = 1 - buf
        drain_gathers(buf)
        pltpu.async_copy(
            rows_v.at[buf],
            out_hbm.at[pl.ds(base + r * ROWS_R, ROWS_R)],
            sem_s.at[buf],
        )
        @pl.when(r >= 1)
        def _():
            pltpu.make_async_copy(
                rows_v.at[other],
                out_hbm.at[pl.ds(base, ROWS_R)],
                sem_s.at[other],
            ).wait()

        @pl.when(r + 1 < ROUNDS)
        def _():
            fire_gathers(r + 1, other)

    def body(i, _):
        round_step(2 * i, 0)
        round_step(2 * i + 1, 1)
        return 0

    lax.fori_loop(0, ROUNDS // 2, body, 0)

    pltpu.make_async_copy(
        rows_v.at[(ROUNDS - 1) % 2],
        out_hbm.at[pl.ds(base, ROWS_R)],
        sem_s.at[(ROUNDS - 1) % 2],
    ).wait()


@jax.jit
def _emb(idx, table):
    mesh = plsc.VectorSubcoreMesh(core_axis_name="c", subcore_axis_name="s")
    f = functools.partial(
        pl.kernel,
        mesh=mesh,
        out_type=jax.ShapeDtypeStruct((B, EMBED), jnp.float32),
        scratch_types=[
            pltpu.VMEM((NCHUNK, CHUNK), jnp.int32),
            pltpu.VMEM((2, ROWS_R, EMBED), jnp.float32),
            pltpu.SemaphoreType.DMA((2,)),
            pltpu.SemaphoreType.DMA((2,)),
        ],
        compiler_params=pltpu.CompilerParams(use_tc_tiling_on_sc=False),
    )(_emb_body)
    return f(idx, table)


def kernel(indices, table):
    idx = indices.reshape(NW, NCHUNK, CHUNK)
    out = _emb(idx, table)
    return out.reshape(BATCH, HIST, EMBED)
